# baseline (speedup 1.0000x reference)
import functools

import jax
import jax.numpy as jnp
from jax import lax
from jax.experimental import pallas as pl
from jax.experimental.pallas import tpu as pltpu
from jax.experimental.pallas import tpu_sc as plsc

B = 1024
L = 20
ITEM_COUNT = 1000
CATE_COUNT = 100
USER_DIM = 128

_NC = 2
_NS = 16
_NW = _NC * _NS
_BPW = B // _NW


def _sc_user_gather(table, idx):
  mesh = plsc.VectorSubcoreMesh(core_axis_name="c", subcore_axis_name="s")

  @functools.partial(
      pl.kernel,
      mesh=mesh,
      out_type=jax.ShapeDtypeStruct((B, USER_DIM), jnp.float32),
      scratch_types=[
          pltpu.VMEM((_BPW,), jnp.int32),
          pltpu.VMEM((_BPW, USER_DIM), jnp.float32),
          pltpu.SemaphoreType.DMA,
      ],
  )
  def k(table_hbm, idx_hbm, out_hbm, idx_v, rows_v, sem):
    wid = lax.axis_index("s") * _NC + lax.axis_index("c")
    base = wid * _BPW
    pltpu.sync_copy(idx_hbm.at[pl.ds(base, _BPW)], idx_v)
    pltpu.async_copy(table_hbm.at[idx_v], rows_v, sem).wait()
    pltpu.sync_copy(rows_v, out_hbm.at[pl.ds(base, _BPW)])

  return k(table, idx)


def _dot_t(x, w):
  return lax.dot_general(x, w, (((1,), (1,)), ((), ())),
                         preferred_element_type=jnp.float32)


def _tc_pool(item_ref, hist_ref, cate_list_ref, item_tab_ref, cate_tab_ref,
             gamma_f_ref, beta_f_ref, w1f_ref, b1_ref, hpart_ref):
  f32 = jnp.float32
  hist = hist_ref[...]
  item = item_ref[...]

  jpos = lax.broadcasted_iota(jnp.int32, (B, L), 1)
  cand = jnp.where((jpos >= 1) & (hist == 0), jpos, L)
  valid_len = jnp.min(cand, axis=1, keepdims=True)
  posmask = (jpos < valid_len).astype(f32)

  iota_col = lax.broadcasted_iota(jnp.int32, (B, 1), 0)
  bad = jnp.where(hist[:, 0:1] == 0, iota_col, B)
  first_bad = jnp.min(bad)
  active = (iota_col < first_bad).astype(f32)

  oc = (cate_list_ref[...] ==
        lax.broadcasted_iota(jnp.int32, (ITEM_COUNT, CATE_COUNT), 1)).astype(f32)
  cate_join = jnp.dot(oc, cate_tab_ref[...], preferred_element_type=f32)
  joined = jnp.concatenate([item_tab_ref[...], cate_join], axis=1)

  bf16 = jnp.bfloat16
  iota16 = lax.broadcasted_iota(jnp.int32, (B, ITEM_COUNT), 1).astype(jnp.int16)
  h16 = hist.astype(jnp.int16)
  m16 = posmask.astype(bf16)
  zero16 = jnp.zeros((B, ITEM_COUNT), bf16)
  counts = zero16
  for l in range(L):
    counts = counts + jnp.where(h16[:, l:l + 1] == iota16,
                                m16[:, l:l + 1], zero16)
  j_hi = joined.astype(bf16)
  j_lo = (joined - j_hi.astype(f32)).astype(bf16)
  pooled_sum = (jnp.dot(counts, j_hi, preferred_element_type=f32)
                + jnp.dot(counts, j_lo, preferred_element_type=f32))
  pooled = (pooled_sum / valid_len.astype(f32)) * active

  oh_item = (item == lax.broadcasted_iota(jnp.int32, (B, ITEM_COUNT), 1)
             ).astype(f32)
  item_join = jnp.dot(oh_item, joined, preferred_element_type=f32)

  feats = jnp.concatenate([item_join, pooled], axis=1)

  mean = jnp.mean(feats, axis=0, keepdims=True)
  d = feats - mean
  var = jnp.mean(d * d, axis=0, keepdims=True)
  h = d * lax.rsqrt(var + 1e-5) * gamma_f_ref[...] + beta_f_ref[...]
  hpart_ref[...] = _dot_t(h, w1f_ref[...]) + b1_ref[...]


def _tc_head(user_emb_ref, hpart_ref, gamma_u_ref, beta_u_ref, w1u_ref,
             a1_ref, w2_ref, b2_ref, a2_ref, w3_ref, b3_ref, out_ref):
  f32 = jnp.float32
  u = user_emb_ref[...]
  mean = jnp.mean(u, axis=0, keepdims=True)
  d = u - mean
  var = jnp.mean(d * d, axis=0, keepdims=True)
  un = d * lax.rsqrt(var + 1e-5) * gamma_u_ref[...] + beta_u_ref[...]

  h = _dot_t(un, w1u_ref[...]) + hpart_ref[...]
  h = jnp.where(h >= 0.0, h, a1_ref[...] * h)
  h = _dot_t(h, w2_ref[...]) + b2_ref[...]
  h = jnp.where(h >= 0.0, h, a2_ref[...] * h)
  h = _dot_t(h, w3_ref[...]) + b3_ref[...]

  h0, h1 = h[:, 0:1], h[:, 1:2]
  m = jnp.maximum(h0, h1)
  e0 = jnp.exp(h0 - m)
  e1 = jnp.exp(h1 - m)
  s = e0 + e1
  out_ref[...] = jnp.concatenate([e0 / s, e1 / s], axis=1)


def kernel(user, item, history, length, cate_list, user_table, item_table,
           cate_table, bn_gamma, bn_beta, W1, b1, a1, W2, b2, a2, W3, b3):
  del length
  i32 = jnp.int32
  user_emb = _sc_user_gather(user_table, user.astype(i32))
  hpart = pl.pallas_call(
      _tc_pool,
      out_shape=jax.ShapeDtypeStruct((B, 200), jnp.float32),
  )(
      item.astype(i32).reshape(B, 1),
      history.astype(i32),
      cate_list.astype(i32).reshape(ITEM_COUNT, 1),
      item_table,
      cate_table,
      bn_gamma[128:],
      bn_beta[128:],
      W1[:, 128:],
      b1,
  )
  out = pl.pallas_call(
      _tc_head,
      out_shape=jax.ShapeDtypeStruct((B, 2), jnp.float32),
  )(
      user_emb,
      hpart,
      bn_gamma[:128],
      bn_beta[:128],
      W1[:, :128],
      a1,
      W2,
      b2,
      a2,
      W3,
      b3,
  )
  return out


# speedup vs baseline: 1.1978x; 1.1978x over previous
import functools

import jax
import jax.numpy as jnp
from jax import lax
from jax.experimental import pallas as pl
from jax.experimental.pallas import tpu as pltpu
from jax.experimental.pallas import tpu_sc as plsc

B = 1024
L = 20
ITEM_COUNT = 1000
CATE_COUNT = 100
USER_DIM = 128

_NC = 2
_NS = 16
_NW = _NC * _NS
_BPW = B // _NW


def _sc_user_gather(table, idx):
  mesh = plsc.VectorSubcoreMesh(core_axis_name="c", subcore_axis_name="s")

  @functools.partial(
      pl.kernel,
      mesh=mesh,
      out_type=jax.ShapeDtypeStruct((B, USER_DIM), jnp.float32),
      scratch_types=[
          pltpu.VMEM((_BPW,), jnp.int32),
          pltpu.VMEM((_BPW, USER_DIM), jnp.float32),
          pltpu.SemaphoreType.DMA,
      ],
  )
  def k(table_hbm, idx_hbm, out_hbm, idx_v, rows_v, sem):
    wid = lax.axis_index("s") * _NC + lax.axis_index("c")
    base = wid * _BPW
    pltpu.sync_copy(idx_hbm.at[pl.ds(base, _BPW)], idx_v)
    pltpu.async_copy(table_hbm.at[idx_v], rows_v, sem).wait()
    pltpu.sync_copy(rows_v, out_hbm.at[pl.ds(base, _BPW)])

  return k(table, idx)


def _dot_t(x, w):
  return lax.dot_general(x, w, (((1,), (1,)), ((), ())),
                         preferred_element_type=jnp.float32)


def _tc_head(user_emb_ref, item_ref, hist_ref, cate_list_ref, item_tab_ref,
             cate_tab_ref, gamma_ref, beta_ref, w1_ref, b1_ref, a1_ref,
             w2_ref, b2_ref, a2_ref, w3_ref, b3_ref, out_ref):
  f32 = jnp.float32
  hist = hist_ref[...]
  item = item_ref[...]

  jpos = lax.broadcasted_iota(jnp.int32, (B, L), 1)
  cand = jnp.where((jpos >= 1) & (hist == 0), jpos, L)
  valid_len = jnp.min(cand, axis=1, keepdims=True)
  posmask = (jpos < valid_len).astype(f32)

  iota_col = lax.broadcasted_iota(jnp.int32, (B, 1), 0)
  bad = jnp.where(hist[:, 0:1] == 0, iota_col, B)
  first_bad = jnp.min(bad)
  active = (iota_col < first_bad).astype(f32)

  oc = (cate_list_ref[...] ==
        lax.broadcasted_iota(jnp.int32, (ITEM_COUNT, CATE_COUNT), 1)).astype(f32)
  cate_join = jnp.dot(oc, cate_tab_ref[...], preferred_element_type=f32)
  joined = jnp.concatenate([item_tab_ref[...], cate_join], axis=1)

  bf16 = jnp.bfloat16
  iota16 = lax.broadcasted_iota(jnp.int32, (B, ITEM_COUNT), 1).astype(jnp.int16)
  h16 = hist.astype(jnp.int16)
  m16 = posmask.astype(bf16)
  zero16 = jnp.zeros((B, ITEM_COUNT), bf16)
  counts = zero16
  for l in range(L):
    counts = counts + jnp.where(h16[:, l:l + 1] == iota16,
                                m16[:, l:l + 1], zero16)
  j_hi = joined.astype(bf16)
  j_lo = (joined - j_hi.astype(f32)).astype(bf16)
  pooled_sum = (jnp.dot(counts, j_hi, preferred_element_type=f32)
                + jnp.dot(counts, j_lo, preferred_element_type=f32))
  pooled = (pooled_sum / valid_len.astype(f32)) * active

  oh_item = (item == lax.broadcasted_iota(jnp.int32, (B, ITEM_COUNT), 1)
             ).astype(f32)
  item_join = jnp.dot(oh_item, joined, preferred_element_type=f32)

  join_emb = jnp.concatenate([user_emb_ref[...], item_join, pooled], axis=1)

  mean = jnp.mean(join_emb, axis=0, keepdims=True)
  d = join_emb - mean
  var = jnp.mean(d * d, axis=0, keepdims=True)
  h = d * lax.rsqrt(var + 1e-5) * gamma_ref[...] + beta_ref[...]

  h = _dot_t(h, w1_ref[...]) + b1_ref[...]
  h = jnp.where(h >= 0.0, h, a1_ref[...] * h)
  h = _dot_t(h, w2_ref[...]) + b2_ref[...]
  h = jnp.where(h >= 0.0, h, a2_ref[...] * h)
  h = _dot_t(h, w3_ref[...]) + b3_ref[...]

  h0, h1 = h[:, 0:1], h[:, 1:2]
  m = jnp.maximum(h0, h1)
  e0 = jnp.exp(h0 - m)
  e1 = jnp.exp(h1 - m)
  s = e0 + e1
  out_ref[...] = jnp.concatenate([e0 / s, e1 / s], axis=1)


def kernel(user, item, history, length, cate_list, user_table, item_table,
           cate_table, bn_gamma, bn_beta, W1, b1, a1, W2, b2, a2, W3, b3):
  del length
  i32 = jnp.int32
  user_emb = _sc_user_gather(user_table, user.astype(i32))
  out = pl.pallas_call(
      _tc_head,
      out_shape=jax.ShapeDtypeStruct((B, 2), jnp.float32),
  )(
      user_emb,
      item.astype(i32).reshape(B, 1),
      history.astype(i32),
      cate_list.astype(i32).reshape(ITEM_COUNT, 1),
      item_table,
      cate_table,
      bn_gamma,
      bn_beta,
      W1,
      b1,
      a1,
      W2,
      b2,
      a2,
      W3,
      b3,
  )
  return out


# speedup vs baseline: 1.2358x; 1.0317x over previous
import functools

import jax
import jax.numpy as jnp
from jax import lax
from jax.experimental import pallas as pl
from jax.experimental.pallas import tpu as pltpu
from jax.experimental.pallas import tpu_sc as plsc

B = 1024
L = 20
ITEM_COUNT = 1000
CATE_COUNT = 100
USER_DIM = 128

_NC = 2
_NS = 16
_NW = _NC * _NS
_BPW = B // _NW


def _sc_user_gather(table, idx):
  mesh = plsc.VectorSubcoreMesh(core_axis_name="c", subcore_axis_name="s")

  @functools.partial(
      pl.kernel,
      mesh=mesh,
      out_type=jax.ShapeDtypeStruct((B, USER_DIM), jnp.float32),
      scratch_types=[
          pltpu.VMEM((_BPW,), jnp.int32),
          pltpu.VMEM((_BPW, USER_DIM), jnp.float32),
          pltpu.SemaphoreType.DMA,
      ],
  )
  def k(table_hbm, idx_hbm, out_hbm, idx_v, rows_v, sem):
    wid = lax.axis_index("s") * _NC + lax.axis_index("c")
    base = wid * _BPW
    pltpu.sync_copy(idx_hbm.at[pl.ds(base, _BPW)], idx_v)
    pltpu.async_copy(table_hbm.at[idx_v], rows_v, sem).wait()
    pltpu.sync_copy(rows_v, out_hbm.at[pl.ds(base, _BPW)])

  return k(table, idx)


def _dot_t(x, w):
  return lax.dot_general(x, w, (((1,), (1,)), ((), ())),
                         preferred_element_type=jnp.float32)


def _tc_head(user_emb_ref, ints_ref, item_tab_ref, cate_tab_ref, vec_ref,
             w1_ref, w2_ref, w3_ref, out_ref):
  f32 = jnp.float32
  ints = ints_ref[...]
  item = ints[:, 0:1]
  hist = ints[:, 1:1 + L]
  cate_list = ints[:ITEM_COUNT, 1 + L:2 + L]
  vec = vec_ref[...]
  gamma, beta = vec[0:1, :], vec[1:2, :]
  b1 = vec[2:3, :200]
  a1 = vec[3:4, 0:1]
  b2 = vec[4:5, :80]
  a2 = vec[5:6, 0:1]
  b3 = vec[6:7, :2]

  jpos = lax.broadcasted_iota(jnp.int32, (B, L), 1)
  cand = jnp.where((jpos >= 1) & (hist == 0), jpos, L)
  valid_len = jnp.min(cand, axis=1, keepdims=True)
  posmask = (jpos < valid_len).astype(f32)

  iota_col = lax.broadcasted_iota(jnp.int32, (B, 1), 0)
  bad = jnp.where(hist[:, 0:1] == 0, iota_col, B)
  first_bad = jnp.min(bad)
  active = (iota_col < first_bad).astype(f32)

  oc = (cate_list ==
        lax.broadcasted_iota(jnp.int32, (ITEM_COUNT, CATE_COUNT), 1)).astype(f32)
  cate_join = jnp.dot(oc, cate_tab_ref[...], preferred_element_type=f32)
  joined = jnp.concatenate([item_tab_ref[...], cate_join], axis=1)

  bf16 = jnp.bfloat16
  iota16 = lax.broadcasted_iota(jnp.int32, (B, ITEM_COUNT), 1).astype(jnp.int16)
  h16 = hist.astype(jnp.int16)
  m16 = posmask.astype(bf16)
  zero16 = jnp.zeros((B, ITEM_COUNT), bf16)
  counts = zero16
  for l in range(L):
    counts = counts + jnp.where(h16[:, l:l + 1] == iota16,
                                m16[:, l:l + 1], zero16)
  j_hi = joined.astype(bf16)
  j_lo = (joined - j_hi.astype(f32)).astype(bf16)
  pooled_sum = (jnp.dot(counts, j_hi, preferred_element_type=f32)
                + jnp.dot(counts, j_lo, preferred_element_type=f32))
  pooled = (pooled_sum / valid_len.astype(f32)) * active

  oh_item = (item == lax.broadcasted_iota(jnp.int32, (B, ITEM_COUNT), 1)
             ).astype(f32)
  item_join = jnp.dot(oh_item, joined, preferred_element_type=f32)

  join_emb = jnp.concatenate([user_emb_ref[...], item_join, pooled], axis=1)

  mean = jnp.mean(join_emb, axis=0, keepdims=True)
  d = join_emb - mean
  var = jnp.mean(d * d, axis=0, keepdims=True)
  h = d * lax.rsqrt(var + 1e-5) * gamma + beta

  h = _dot_t(h, w1_ref[...]) + b1
  h = jnp.where(h >= 0.0, h, a1 * h)
  h = _dot_t(h, w2_ref[...]) + b2
  h = jnp.where(h >= 0.0, h, a2 * h)
  h = _dot_t(h, w3_ref[...]) + b3

  h0, h1 = h[:, 0:1], h[:, 1:2]
  m = jnp.maximum(h0, h1)
  e0 = jnp.exp(h0 - m)
  e1 = jnp.exp(h1 - m)
  s = e0 + e1
  out_ref[...] = jnp.concatenate([e0 / s, e1 / s], axis=1)


def kernel(user, item, history, length, cate_list, user_table, item_table,
           cate_table, bn_gamma, bn_beta, W1, b1, a1, W2, b2, a2, W3, b3):
  del length
  i32 = jnp.int32
  f32 = jnp.float32
  user_emb = _sc_user_gather(user_table, user.astype(i32))

  cate_pad = jnp.pad(cate_list.astype(i32), (0, B - ITEM_COUNT))
  ints = jnp.concatenate(
      [item.astype(i32).reshape(B, 1), history.astype(i32),
       cate_pad.reshape(B, 1)], axis=1)

  def _row(v):
    v = v.astype(f32)
    return jnp.pad(v, (0, 384 - v.shape[0])).reshape(1, 384)

  vec = jnp.concatenate(
      [_row(bn_gamma), _row(bn_beta), _row(b1), _row(a1), _row(b2), _row(a2),
       _row(b3)], axis=0)

  out = pl.pallas_call(
      _tc_head,
      out_shape=jax.ShapeDtypeStruct((B, 2), jnp.float32),
  )(
      user_emb,
      ints,
      item_table,
      cate_table,
      vec,
      W1,
      W2,
      W3,
  )
  return out
